# baseline (speedup 1.0000x reference)
import functools

import jax
import jax.numpy as jnp
from jax import lax
from jax.experimental import pallas as pl
from jax.experimental.pallas import tpu as pltpu
from jax.experimental.pallas import tpu_sc as plsc

_TWO_PI = 2.0 * 3.14159
_ECC = 3.0
_LAM1 = 2.0 / (1.0 + _ECC)
_LAM2 = 2.0 - _LAM1

_B, _H, _GY, _GX = 32, 8, 160, 160
_GXP = 256
_PLANEP = _GY * _GXP
_IMG_STRIDE = _H * _PLANEP
_NOBJ = 262144
_NA = 1048576
_NPP = _B * _H * _PLANEP

_NW = 32
_NC = 2
_PER_W = _NA // _NW
_CHUNK = 8192
_SUB = _CHUNK // 128
_NCHUNK = _PER_W // _CHUNK
_ROWS_W = _PER_W // 128


def _pack_words(a, b):
    b1 = jax.lax.bitcast_convert_type(a.astype(jnp.bfloat16), jnp.uint16)
    b2 = jax.lax.bitcast_convert_type(b.astype(jnp.bfloat16), jnp.uint16)
    return b1.astype(jnp.int32) | (b2.astype(jnp.int32) << 16)


def _trig_body(rot_ref, hb_ref, out_ref):
    rad = rot_ref[...] * _TWO_PI
    hb = hb_ref[...]
    out_ref[...] = _pack_words(jnp.sin(rad) * hb, jnp.cos(rad) * hb)


def _trig_tables(rotation, has_rotation):
    rot2 = rotation.reshape(_NOBJ // 128, 128)
    hb2 = has_rotation.astype(jnp.float32).reshape(_NOBJ // 128, 128)
    tp = pl.pallas_call(
        _trig_body,
        out_shape=jax.ShapeDtypeStruct((_NOBJ // 128, 128), jnp.int32),
    )(rot2, hb2)
    return tp.reshape(_NOBJ)


def _p_pack_body(p_ref, out_ref):
    packed = _pack_words(p_ref[0, 0, 0], p_ref[0, 0, 1])
    pad = jnp.zeros((_GY, _GXP - _GX), jnp.int32)
    out_ref[0, 0] = jnp.concatenate([packed, pad], axis=1)


def _pack_predictions(p):
    out = pl.pallas_call(
        _p_pack_body,
        grid=(_B, _H),
        in_specs=[pl.BlockSpec((1, 1, 2, _GY, _GX),
                               lambda b, h: (b, h, 0, 0, 0))],
        out_specs=pl.BlockSpec((1, 1, _GY, _GXP),
                               lambda b, h: (b, h, 0, 0)),
        out_shape=jax.ShapeDtypeStruct((_B, _H, _GY, _GXP), jnp.int32),
    )(p)
    return out.reshape(_NPP)


@functools.partial(
    pl.kernel,
    out_type=jax.ShapeDtypeStruct((_NW, 16), jnp.float32),
    mesh=plsc.VectorSubcoreMesh(core_axis_name="c", subcore_axis_name="s"),
    compiler_params=pltpu.CompilerParams(needs_layout_passes=False),
    scratch_types=[
        pltpu.VMEM((_SUB, 128), jnp.int32),
        pltpu.VMEM((_SUB, 128), jnp.int32),
        pltpu.VMEM((_SUB, 128), jnp.int32),
        pltpu.VMEM((_SUB, 128), jnp.int32),
        pltpu.VMEM((_SUB, 128), jnp.int32),
        pltpu.VMEM((_SUB, 128), jnp.int32),
        pltpu.VMEM((_SUB, 128), jnp.int32),
        pltpu.VMEM((_SUB, 128), jnp.int32),
        pltpu.VMEM((16,), jnp.float32),
        pltpu.SemaphoreType.DMA,
    ],
)
def _sc_loss(pp_hbm, tp_hbm, img_hbm, head_hbm, gy_hbm, gx_hbm,
             obj_hbm, out_hbm,
             img_v, head_v, gy_v, gx_v, obj_v, fr_v,
             praw_v, traw_v, res_v, sem):
    cid = lax.axis_index("c")
    sid = lax.axis_index("s")
    wid = sid * _NC + cid
    row0 = wid * _ROWS_W

    def chunk_body(t, carry):
        acc1, acc2 = carry
        r0 = row0 + t * _SUB
        c1 = pltpu.async_copy(img_hbm.at[pl.ds(r0, _SUB)], img_v, sem)
        c2 = pltpu.async_copy(head_hbm.at[pl.ds(r0, _SUB)], head_v, sem)
        c3 = pltpu.async_copy(gy_hbm.at[pl.ds(r0, _SUB)], gy_v, sem)
        c4 = pltpu.async_copy(gx_hbm.at[pl.ds(r0, _SUB)], gx_v, sem)
        c5 = pltpu.async_copy(obj_hbm.at[pl.ds(r0, _SUB)], obj_v, sem)
        c1.wait(); c2.wait(); c3.wait(); c4.wait(); c5.wait()

        def idx_row(r, u):
            for k in range(8):
                sl = pl.ds(k * 16, 16)
                fr_v[r, sl] = (img_v[r, sl] * _IMG_STRIDE
                               + head_v[r, sl] * _PLANEP
                               + gy_v[r, sl] * _GXP + gx_v[r, sl])
            return u
        lax.fori_loop(0, _SUB, idx_row, 0)

        def gather_row(r, u):
            pltpu.async_copy(pp_hbm.at[fr_v.at[r]], praw_v.at[r], sem)
            pltpu.async_copy(tp_hbm.at[obj_v.at[r]], traw_v.at[r], sem)
            return u
        lax.fori_loop(0, _SUB, gather_row, 0)

        def drain_row(r, u):
            pltpu.make_async_copy(pp_hbm.at[pl.ds(0, 128)], praw_v.at[r], sem).wait()
            pltpu.make_async_copy(pp_hbm.at[pl.ds(0, 128)], traw_v.at[r], sem).wait()
            return u
        lax.fori_loop(0, _SUB, drain_row, 0)

        hi_mask = jnp.full((16,), -65536, jnp.int32)

        def comp_row(r, cc):
            a1, a2 = cc
            for k in range(8):
                sl = pl.ds(k * 16, 16)
                pu = praw_v[r, sl]
                tu = traw_v[r, sl]
                p1 = plsc.bitcast(pu << 16, jnp.float32)
                p2 = plsc.bitcast(pu & hi_mask, jnp.float32)
                sb = plsc.bitcast(tu << 16, jnp.float32)
                cb = plsc.bitcast(tu & hi_mask, jnp.float32)
                bf = sb * sb + cb * cb
                t1 = p1 * sb + p2 * cb - bf
                t2 = p1 * cb - p2 * sb
                a1 = a1 + t1 * t1
                a2 = a2 + t2 * t2
            return (a1, a2)
        return lax.fori_loop(0, _SUB, comp_row, (acc1, acc2))

    zero = jnp.zeros((16,), jnp.float32)
    acc1, acc2 = lax.fori_loop(0, _NCHUNK, chunk_body, (zero, zero))
    res_v[...] = acc1 * _LAM1 + acc2 * _LAM2
    pltpu.sync_copy(res_v, out_hbm.at[wid])


def kernel(post_activation_sincos, rotation, has_rotation, object_idxs,
           img_idxs, head_idxs, grid_y_idxs, grid_x_idxs):
    tpack = _trig_tables(rotation, has_rotation)
    ppack = _pack_predictions(post_activation_sincos)
    img2 = img_idxs.reshape(_NA // 128, 128)
    head2 = head_idxs.reshape(_NA // 128, 128)
    gy2 = grid_y_idxs.reshape(_NA // 128, 128)
    gx2 = grid_x_idxs.reshape(_NA // 128, 128)
    obj2 = object_idxs.reshape(_NA // 128, 128)
    partials = _sc_loss(ppack, tpack, img2, head2, gy2, gx2, obj2)
    return jnp.sum(partials)


# speedup vs baseline: 1.0918x; 1.0918x over previous
import functools

import jax
import jax.numpy as jnp
from jax import lax
from jax.experimental import pallas as pl
from jax.experimental.pallas import tpu as pltpu
from jax.experimental.pallas import tpu_sc as plsc

_TWO_PI = 2.0 * 3.14159
_ECC = 3.0
_LAM1 = 2.0 / (1.0 + _ECC)
_LAM2 = 2.0 - _LAM1

_B, _H, _GY, _GX = 32, 8, 160, 160
_GXP = 256
_PLANEP = _GY * _GXP
_IMG_STRIDE = _H * _PLANEP
_NOBJ = 262144
_NA = 1048576
_NPP = _B * _H * _PLANEP

_NW = 32
_NC = 2
_PER_W = _NA // _NW
_CHUNK = 8192
_SUB = _CHUNK // 128
_NCHUNK = _PER_W // _CHUNK
_ROWS_W = _PER_W // 128


def _pack_words(a, b):
    b1 = jax.lax.bitcast_convert_type(a.astype(jnp.bfloat16), jnp.uint16)
    b2 = jax.lax.bitcast_convert_type(b.astype(jnp.bfloat16), jnp.uint16)
    return b1.astype(jnp.int32) | (b2.astype(jnp.int32) << 16)


def _trig_body(rot_ref, hb_ref, out_ref):
    rad = rot_ref[...] * _TWO_PI
    hb = hb_ref[...]
    out_ref[...] = _pack_words(jnp.sin(rad) * hb, jnp.cos(rad) * hb)


def _trig_tables(rotation, has_rotation):
    rot2 = rotation.reshape(_NOBJ // 128, 128)
    hb2 = has_rotation.astype(jnp.float32).reshape(_NOBJ // 128, 128)
    tp = pl.pallas_call(
        _trig_body,
        out_shape=jax.ShapeDtypeStruct((_NOBJ // 128, 128), jnp.int32),
    )(rot2, hb2)
    return tp.reshape(_NOBJ)


def _p_pack_body(p_ref, out_ref):
    packed = _pack_words(p_ref[0, 0, 0], p_ref[0, 0, 1])
    a = packed[:, :128]
    b = jnp.concatenate(
        [packed[:, 128:], jnp.zeros((_GY, 96), jnp.int32)], axis=1)
    a3 = a.reshape(_GY // 8, 1, 8, 128)
    b3 = b.reshape(_GY // 8, 1, 8, 128)
    out_ref[0, 0] = jnp.concatenate([a3, b3], axis=1).reshape(2 * _GY, 128)


def _pack_predictions(p):
    out = pl.pallas_call(
        _p_pack_body,
        grid=(_B, _H),
        in_specs=[pl.BlockSpec((1, 1, 2, _GY, _GX),
                               lambda b, h: (b, h, 0, 0, 0))],
        out_specs=pl.BlockSpec((1, 1, 2 * _GY, 128),
                               lambda b, h: (b, h, 0, 0)),
        out_shape=jax.ShapeDtypeStruct((_B, _H, 2 * _GY, 128), jnp.int32),
    )(p)
    return out.reshape(_NPP)


@functools.partial(
    pl.kernel,
    out_type=jax.ShapeDtypeStruct((_NW, 16), jnp.float32),
    mesh=plsc.VectorSubcoreMesh(core_axis_name="c", subcore_axis_name="s"),
    compiler_params=pltpu.CompilerParams(needs_layout_passes=False),
    scratch_types=[
        pltpu.VMEM((_SUB, 128), jnp.int32),
        pltpu.VMEM((_SUB, 128), jnp.int32),
        pltpu.VMEM((_SUB, 128), jnp.int32),
        pltpu.VMEM((_SUB, 128), jnp.int32),
        pltpu.VMEM((_SUB, 128), jnp.int32),
        pltpu.VMEM((_SUB, 128), jnp.int32),
        pltpu.VMEM((_SUB, 128), jnp.int32),
        pltpu.VMEM((_SUB, 128), jnp.int32),
        pltpu.VMEM((16,), jnp.float32),
        pltpu.SemaphoreType.DMA,
    ],
)
def _sc_loss(pp_hbm, tp_hbm, img_hbm, head_hbm, gy_hbm, gx_hbm,
             obj_hbm, out_hbm,
             img_v, head_v, gy_v, gx_v, obj_v, fr_v,
             praw_v, traw_v, res_v, sem):
    cid = lax.axis_index("c")
    sid = lax.axis_index("s")
    wid = sid * _NC + cid
    row0 = wid * _ROWS_W

    def chunk_body(t, carry):
        acc1, acc2 = carry
        r0 = row0 + t * _SUB
        c1 = pltpu.async_copy(img_hbm.at[pl.ds(r0, _SUB)], img_v, sem)
        c2 = pltpu.async_copy(head_hbm.at[pl.ds(r0, _SUB)], head_v, sem)
        c3 = pltpu.async_copy(gy_hbm.at[pl.ds(r0, _SUB)], gy_v, sem)
        c4 = pltpu.async_copy(gx_hbm.at[pl.ds(r0, _SUB)], gx_v, sem)
        c5 = pltpu.async_copy(obj_hbm.at[pl.ds(r0, _SUB)], obj_v, sem)
        c1.wait(); c2.wait(); c3.wait(); c4.wait(); c5.wait()

        def idx_row(r, u):
            for k in range(8):
                sl = pl.ds(k * 16, 16)
                gy = gy_v[r, sl]
                gx = gx_v[r, sl]
                fr_v[r, sl] = (img_v[r, sl] * _IMG_STRIDE
                               + head_v[r, sl] * _PLANEP
                               + ((gy >> 3) << 11) + ((gx >> 7) << 10)
                               + ((gy & 7) << 7) + (gx & 127))
            return u
        lax.fori_loop(0, _SUB, idx_row, 0)

        def gather_row(r, u):
            pltpu.async_copy(pp_hbm.at[fr_v.at[r]], praw_v.at[r], sem)
            pltpu.async_copy(tp_hbm.at[obj_v.at[r]], traw_v.at[r], sem)
            return u
        lax.fori_loop(0, _SUB, gather_row, 0)

        def drain_row(r, u):
            pltpu.make_async_copy(pp_hbm.at[pl.ds(0, 128)], praw_v.at[r], sem).wait()
            pltpu.make_async_copy(pp_hbm.at[pl.ds(0, 128)], traw_v.at[r], sem).wait()
            return u
        lax.fori_loop(0, _SUB, drain_row, 0)

        hi_mask = jnp.full((16,), -65536, jnp.int32)

        def comp_row(r, cc):
            a1, a2 = cc
            for k in range(8):
                sl = pl.ds(k * 16, 16)
                pu = praw_v[r, sl]
                tu = traw_v[r, sl]
                p1 = plsc.bitcast(pu << 16, jnp.float32)
                p2 = plsc.bitcast(pu & hi_mask, jnp.float32)
                sb = plsc.bitcast(tu << 16, jnp.float32)
                cb = plsc.bitcast(tu & hi_mask, jnp.float32)
                bf = sb * sb + cb * cb
                t1 = p1 * sb + p2 * cb - bf
                t2 = p1 * cb - p2 * sb
                a1 = a1 + t1 * t1
                a2 = a2 + t2 * t2
            return (a1, a2)
        return lax.fori_loop(0, _SUB, comp_row, (acc1, acc2))

    zero = jnp.zeros((16,), jnp.float32)
    acc1, acc2 = lax.fori_loop(0, _NCHUNK, chunk_body, (zero, zero))
    res_v[...] = acc1 * _LAM1 + acc2 * _LAM2
    pltpu.sync_copy(res_v, out_hbm.at[wid])


def kernel(post_activation_sincos, rotation, has_rotation, object_idxs,
           img_idxs, head_idxs, grid_y_idxs, grid_x_idxs):
    tpack = _trig_tables(rotation, has_rotation)
    ppack = _pack_predictions(post_activation_sincos)
    img2 = img_idxs.reshape(_NA // 128, 128)
    head2 = head_idxs.reshape(_NA // 128, 128)
    gy2 = grid_y_idxs.reshape(_NA // 128, 128)
    gx2 = grid_x_idxs.reshape(_NA // 128, 128)
    obj2 = object_idxs.reshape(_NA // 128, 128)
    partials = _sc_loss(ppack, tpack, img2, head2, gy2, gx2, obj2)
    return jnp.sum(partials)


# speedup vs baseline: 1.7785x; 1.6290x over previous
import functools

import jax
import jax.numpy as jnp
from jax import lax
from jax.experimental import pallas as pl
from jax.experimental.pallas import tpu as pltpu
from jax.experimental.pallas import tpu_sc as plsc

_TWO_PI = 2.0 * 3.14159
_ECC = 3.0
_LAM1 = 2.0 / (1.0 + _ECC)
_LAM2 = 2.0 - _LAM1

_B, _H, _GY, _GX = 32, 8, 160, 160
_GXP = 256
_PLANEP = _GY * _GXP
_IMG_STRIDE = _H * _PLANEP
_NOBJ = 262144
_NA = 1048576
_NPP = _B * _H * _PLANEP

_NW = 32
_NC = 2
_PER_W = _NA // _NW
_CHUNK = 8192
_SUB = _CHUNK // 128
_NCHUNK = _PER_W // _CHUNK
_ROWS_W = _PER_W // 128


def _pack_words(a, b):
    b1 = jax.lax.bitcast_convert_type(a.astype(jnp.bfloat16), jnp.uint16)
    b2 = jax.lax.bitcast_convert_type(b.astype(jnp.bfloat16), jnp.uint16)
    return b1.astype(jnp.int32) | (b2.astype(jnp.int32) << 16)


def _trig_body(rot_ref, hb_ref, out_ref):
    rad = rot_ref[...] * _TWO_PI
    hb = hb_ref[...]
    out_ref[...] = _pack_words(jnp.sin(rad) * hb, jnp.cos(rad) * hb)


def _trig_tables(rotation, has_rotation):
    rot2 = rotation.reshape(_NOBJ // 128, 128)
    hb2 = has_rotation.astype(jnp.float32).reshape(_NOBJ // 128, 128)
    tp = pl.pallas_call(
        _trig_body,
        out_shape=jax.ShapeDtypeStruct((_NOBJ // 128, 128), jnp.int32),
    )(rot2, hb2)
    return tp.reshape(_NOBJ)


def _p_pack_body(p_ref, out_ref):
    for h in range(_H):
        packed = _pack_words(p_ref[0, h, 0], p_ref[0, h, 1])
        a = packed[:, :128]
        b = jnp.concatenate(
            [packed[:, 128:], jnp.zeros((_GY, 96), jnp.int32)], axis=1)
        a3 = a.reshape(_GY // 8, 1, 8, 128)
        b3 = b.reshape(_GY // 8, 1, 8, 128)
        out_ref[0, h] = jnp.concatenate([a3, b3], axis=1).reshape(2 * _GY, 128)


def _pack_predictions(p):
    out = pl.pallas_call(
        _p_pack_body,
        grid=(_B,),
        in_specs=[pl.BlockSpec((1, _H, 2, _GY, _GX),
                               lambda b: (b, 0, 0, 0, 0))],
        out_specs=pl.BlockSpec((1, _H, 2 * _GY, 128),
                               lambda b: (b, 0, 0, 0)),
        out_shape=jax.ShapeDtypeStruct((_B, _H, 2 * _GY, 128), jnp.int32),
    )(p)
    return out.reshape(_NPP)


@functools.partial(
    pl.kernel,
    out_type=jax.ShapeDtypeStruct((_NW, 16), jnp.float32),
    mesh=plsc.VectorSubcoreMesh(core_axis_name="c", subcore_axis_name="s"),
    compiler_params=pltpu.CompilerParams(needs_layout_passes=False),
    scratch_types=[
        pltpu.VMEM((_SUB, 128), jnp.int32),
        pltpu.VMEM((_SUB, 128), jnp.int32),
        pltpu.VMEM((_SUB, 128), jnp.int32),
        pltpu.VMEM((_SUB, 128), jnp.int32),
        pltpu.VMEM((_SUB, 128), jnp.int32),
        pltpu.VMEM((_SUB, 128), jnp.int32),
        pltpu.VMEM((_SUB, 128), jnp.int32),
        pltpu.VMEM((_SUB, 128), jnp.int32),
        pltpu.VMEM((16,), jnp.float32),
        pltpu.SemaphoreType.DMA,
    ],
)
def _sc_loss(pp_hbm, tp_hbm, img_hbm, head_hbm, gy_hbm, gx_hbm,
             obj_hbm, out_hbm,
             img_v, head_v, gy_v, gx_v, obj_v, fr_v,
             praw_v, traw_v, res_v, sem):
    cid = lax.axis_index("c")
    sid = lax.axis_index("s")
    wid = sid * _NC + cid
    row0 = wid * _ROWS_W

    def chunk_body(t, carry):
        acc1, acc2 = carry
        r0 = row0 + t * _SUB
        c1 = pltpu.async_copy(img_hbm.at[pl.ds(r0, _SUB)], img_v, sem)
        c2 = pltpu.async_copy(head_hbm.at[pl.ds(r0, _SUB)], head_v, sem)
        c3 = pltpu.async_copy(gy_hbm.at[pl.ds(r0, _SUB)], gy_v, sem)
        c4 = pltpu.async_copy(gx_hbm.at[pl.ds(r0, _SUB)], gx_v, sem)
        c5 = pltpu.async_copy(obj_hbm.at[pl.ds(r0, _SUB)], obj_v, sem)
        c1.wait(); c2.wait(); c3.wait(); c4.wait(); c5.wait()

        def idx_row(r, u):
            for k in range(8):
                sl = pl.ds(k * 16, 16)
                gy = gy_v[r, sl]
                gx = gx_v[r, sl]
                fr_v[r, sl] = (img_v[r, sl] * _IMG_STRIDE
                               + head_v[r, sl] * _PLANEP
                               + ((gy >> 3) << 11) + ((gx >> 7) << 10)
                               + ((gy & 7) << 7) + (gx & 127))
            return u
        lax.fori_loop(0, _SUB, idx_row, 0)

        def gather_row(r, u):
            pltpu.async_copy(pp_hbm.at[fr_v.at[r]], praw_v.at[r], sem)
            pltpu.async_copy(tp_hbm.at[obj_v.at[r]], traw_v.at[r], sem)
            return u
        lax.fori_loop(0, _SUB, gather_row, 0)

        def drain_row(r, u):
            pltpu.make_async_copy(pp_hbm.at[pl.ds(0, 128)], praw_v.at[r], sem).wait()
            pltpu.make_async_copy(pp_hbm.at[pl.ds(0, 128)], traw_v.at[r], sem).wait()
            return u
        lax.fori_loop(0, _SUB, drain_row, 0)

        hi_mask = jnp.full((16,), -65536, jnp.int32)

        def comp_row(r, cc):
            a1, a2 = cc
            for k in range(8):
                sl = pl.ds(k * 16, 16)
                pu = praw_v[r, sl]
                tu = traw_v[r, sl]
                p1 = plsc.bitcast(pu << 16, jnp.float32)
                p2 = plsc.bitcast(pu & hi_mask, jnp.float32)
                sb = plsc.bitcast(tu << 16, jnp.float32)
                cb = plsc.bitcast(tu & hi_mask, jnp.float32)
                bf = sb * sb + cb * cb
                t1 = p1 * sb + p2 * cb - bf
                t2 = p1 * cb - p2 * sb
                a1 = a1 + t1 * t1
                a2 = a2 + t2 * t2
            return (a1, a2)
        return lax.fori_loop(0, _SUB, comp_row, (acc1, acc2))

    zero = jnp.zeros((16,), jnp.float32)
    acc1, acc2 = lax.fori_loop(0, _NCHUNK, chunk_body, (zero, zero))
    res_v[...] = acc1 * _LAM1 + acc2 * _LAM2
    pltpu.sync_copy(res_v, out_hbm.at[wid])


def kernel(post_activation_sincos, rotation, has_rotation, object_idxs,
           img_idxs, head_idxs, grid_y_idxs, grid_x_idxs):
    tpack = _trig_tables(rotation, has_rotation)
    ppack = _pack_predictions(post_activation_sincos)
    img2 = img_idxs.reshape(_NA // 128, 128)
    head2 = head_idxs.reshape(_NA // 128, 128)
    gy2 = grid_y_idxs.reshape(_NA // 128, 128)
    gx2 = grid_x_idxs.reshape(_NA // 128, 128)
    obj2 = object_idxs.reshape(_NA // 128, 128)
    partials = _sc_loss(ppack, tpack, img2, head2, gy2, gx2, obj2)
    return jnp.sum(partials)


# speedup vs baseline: 1.8639x; 1.0480x over previous
import functools

import jax
import jax.numpy as jnp
from jax import lax
from jax.experimental import pallas as pl
from jax.experimental.pallas import tpu as pltpu
from jax.experimental.pallas import tpu_sc as plsc

_TWO_PI = 2.0 * 3.14159
_ECC = 3.0
_LAM1 = 2.0 / (1.0 + _ECC)
_LAM2 = 2.0 - _LAM1

_B, _H, _GY, _GX = 32, 8, 160, 160
_GXP = 256
_PLANEP = _GY * _GXP
_IMG_STRIDE = _H * _PLANEP
_NOBJ = 262144
_NA = 1048576
_NPP = _B * _H * _PLANEP

_NW = 32
_NC = 2
_PER_W = _NA // _NW
_CHUNK = 16384
_SUB = _CHUNK // 128
_NCHUNK = _PER_W // _CHUNK
_ROWS_W = _PER_W // 128


def _pack_words(a, b):
    b1 = jax.lax.bitcast_convert_type(a.astype(jnp.bfloat16), jnp.uint16)
    b2 = jax.lax.bitcast_convert_type(b.astype(jnp.bfloat16), jnp.uint16)
    return b1.astype(jnp.int32) | (b2.astype(jnp.int32) << 16)


def _trig_body(rot_ref, hb_ref, out_ref):
    rad = rot_ref[...] * _TWO_PI
    hb = hb_ref[...]
    out_ref[...] = _pack_words(jnp.sin(rad) * hb, jnp.cos(rad) * hb)


def _trig_tables(rotation, has_rotation):
    rot2 = rotation.reshape(_NOBJ // 128, 128)
    hb2 = has_rotation.astype(jnp.float32).reshape(_NOBJ // 128, 128)
    tp = pl.pallas_call(
        _trig_body,
        out_shape=jax.ShapeDtypeStruct((_NOBJ // 128, 128), jnp.int32),
    )(rot2, hb2)
    return tp.reshape(_NOBJ)


def _p_pack_body(p_ref, out_ref):
    for h in range(_H):
        packed = _pack_words(p_ref[0, h, 0], p_ref[0, h, 1])
        a = packed[:, :128]
        b = jnp.concatenate(
            [packed[:, 128:], jnp.zeros((_GY, 96), jnp.int32)], axis=1)
        a3 = a.reshape(_GY // 8, 1, 8, 128)
        b3 = b.reshape(_GY // 8, 1, 8, 128)
        out_ref[0, h] = jnp.concatenate([a3, b3], axis=1).reshape(2 * _GY, 128)


def _pack_predictions(p):
    out = pl.pallas_call(
        _p_pack_body,
        grid=(_B,),
        in_specs=[pl.BlockSpec((1, _H, 2, _GY, _GX),
                               lambda b: (b, 0, 0, 0, 0))],
        out_specs=pl.BlockSpec((1, _H, 2 * _GY, 128),
                               lambda b: (b, 0, 0, 0)),
        out_shape=jax.ShapeDtypeStruct((_B, _H, 2 * _GY, 128), jnp.int32),
    )(p)
    return out.reshape(_NPP)


@functools.partial(
    pl.kernel,
    out_type=jax.ShapeDtypeStruct((_NA // 128, 128), jnp.int32),
    mesh=plsc.VectorSubcoreMesh(core_axis_name="c", subcore_axis_name="s"),
    compiler_params=pltpu.CompilerParams(needs_layout_passes=False),
    scratch_types=[
        pltpu.VMEM((_ROWS_W, 128), jnp.int32),
        pltpu.VMEM((_ROWS_W, 128), jnp.int32),
        pltpu.SemaphoreType.DMA,
    ],
)
def _sc_tgather(tp_hbm, obj_hbm, out_hbm, obj_v, g_v, sem):
    cid = lax.axis_index("c")
    sid = lax.axis_index("s")
    wid = sid * _NC + cid
    row0 = wid * _ROWS_W
    pltpu.async_copy(obj_hbm.at[pl.ds(row0, _ROWS_W)], obj_v, sem).wait()

    def gather_row(r, u):
        pltpu.async_copy(tp_hbm.at[obj_v.at[r]], g_v.at[r], sem)
        return u
    lax.fori_loop(0, _ROWS_W, gather_row, 0)

    def drain_row(r, u):
        pltpu.make_async_copy(tp_hbm.at[pl.ds(0, 128)], g_v.at[r], sem).wait()
        return u
    lax.fori_loop(0, _ROWS_W, drain_row, 0)
    pltpu.sync_copy(g_v, out_hbm.at[pl.ds(row0, _ROWS_W)])


@functools.partial(
    pl.kernel,
    out_type=jax.ShapeDtypeStruct((_NW, 16), jnp.float32),
    mesh=plsc.VectorSubcoreMesh(core_axis_name="c", subcore_axis_name="s"),
    compiler_params=pltpu.CompilerParams(needs_layout_passes=False),
    scratch_types=[
        pltpu.VMEM((_SUB, 128), jnp.int32),
        pltpu.VMEM((_SUB, 128), jnp.int32),
        pltpu.VMEM((_SUB, 128), jnp.int32),
        pltpu.VMEM((_SUB, 128), jnp.int32),
        pltpu.VMEM((_SUB, 128), jnp.int32),
        pltpu.VMEM((_SUB, 128), jnp.int32),
        pltpu.VMEM((_SUB, 128), jnp.int32),
        pltpu.VMEM((16,), jnp.float32),
        pltpu.SemaphoreType.DMA,
    ],
)
def _sc_loss(pp_hbm, tw_hbm, img_hbm, head_hbm, gy_hbm, gx_hbm,
             out_hbm,
             img_v, head_v, gy_v, gx_v, fr_v,
             praw_v, traw_v, res_v, sem):
    cid = lax.axis_index("c")
    sid = lax.axis_index("s")
    wid = sid * _NC + cid
    row0 = wid * _ROWS_W

    def chunk_body(t, carry):
        acc1, acc2 = carry
        r0 = row0 + t * _SUB
        c1 = pltpu.async_copy(img_hbm.at[pl.ds(r0, _SUB)], img_v, sem)
        c2 = pltpu.async_copy(head_hbm.at[pl.ds(r0, _SUB)], head_v, sem)
        c3 = pltpu.async_copy(gy_hbm.at[pl.ds(r0, _SUB)], gy_v, sem)
        c4 = pltpu.async_copy(gx_hbm.at[pl.ds(r0, _SUB)], gx_v, sem)
        c5 = pltpu.async_copy(tw_hbm.at[pl.ds(r0, _SUB)], traw_v, sem)
        c1.wait(); c2.wait(); c3.wait(); c4.wait(); c5.wait()

        def idx_row(r, u):
            for k in range(8):
                sl = pl.ds(k * 16, 16)
                gy = gy_v[r, sl]
                gx = gx_v[r, sl]
                fr_v[r, sl] = (img_v[r, sl] * _IMG_STRIDE
                               + head_v[r, sl] * _PLANEP
                               + ((gy >> 3) << 11) + ((gx >> 7) << 10)
                               + ((gy & 7) << 7) + (gx & 127))
            return u
        lax.fori_loop(0, _SUB, idx_row, 0)

        def gather_row(r, u):
            pltpu.async_copy(pp_hbm.at[fr_v.at[r]], praw_v.at[r], sem)
            return u
        lax.fori_loop(0, _SUB, gather_row, 0)

        def drain_row(r, u):
            pltpu.make_async_copy(pp_hbm.at[pl.ds(0, 128)], praw_v.at[r], sem).wait()
            return u
        lax.fori_loop(0, _SUB, drain_row, 0)

        hi_mask = jnp.full((16,), -65536, jnp.int32)

        def comp_row(r, cc):
            a1, a2 = cc
            for k in range(8):
                sl = pl.ds(k * 16, 16)
                pu = praw_v[r, sl]
                tu = traw_v[r, sl]
                p1 = plsc.bitcast(pu << 16, jnp.float32)
                p2 = plsc.bitcast(pu & hi_mask, jnp.float32)
                sb = plsc.bitcast(tu << 16, jnp.float32)
                cb = plsc.bitcast(tu & hi_mask, jnp.float32)
                bf = sb * sb + cb * cb
                t1 = p1 * sb + p2 * cb - bf
                t2 = p1 * cb - p2 * sb
                a1 = a1 + t1 * t1
                a2 = a2 + t2 * t2
            return (a1, a2)
        return lax.fori_loop(0, _SUB, comp_row, (acc1, acc2))

    zero = jnp.zeros((16,), jnp.float32)
    acc1, acc2 = lax.fori_loop(0, _NCHUNK, chunk_body, (zero, zero))
    res_v[...] = acc1 * _LAM1 + acc2 * _LAM2
    pltpu.sync_copy(res_v, out_hbm.at[wid])


def kernel(post_activation_sincos, rotation, has_rotation, object_idxs,
           img_idxs, head_idxs, grid_y_idxs, grid_x_idxs):
    tpack = _trig_tables(rotation, has_rotation)
    obj2 = object_idxs.reshape(_NA // 128, 128)
    tw = _sc_tgather(tpack, obj2)
    ppack = _pack_predictions(post_activation_sincos)
    img2 = img_idxs.reshape(_NA // 128, 128)
    head2 = head_idxs.reshape(_NA // 128, 128)
    gy2 = grid_y_idxs.reshape(_NA // 128, 128)
    gx2 = grid_x_idxs.reshape(_NA // 128, 128)
    partials = _sc_loss(ppack, tw, img2, head2, gy2, gx2)
    return jnp.sum(partials)


# speedup vs baseline: 1.9366x; 1.0390x over previous
import functools

import jax
import jax.numpy as jnp
from jax import lax
from jax.experimental import pallas as pl
from jax.experimental.pallas import tpu as pltpu
from jax.experimental.pallas import tpu_sc as plsc

_TWO_PI = 2.0 * 3.14159
_ECC = 3.0
_LAM1 = 2.0 / (1.0 + _ECC)
_LAM2 = 2.0 - _LAM1

_B, _H, _GY, _GX = 32, 8, 160, 160
_GXP = 256
_PLANEP = _GY * _GXP
_IMG_STRIDE = _H * _PLANEP
_NOBJ = 262144
_NA = 1048576
_NPP = _B * _H * _PLANEP

_NW = 32
_NC = 2
_PER_W = _NA // _NW
_CHUNK = 8192
_SUB = _CHUNK // 128
_NCHUNK = _PER_W // _CHUNK
_ROWS_W = _PER_W // 128


def _pack_words(a, b):
    b1 = jax.lax.bitcast_convert_type(a.astype(jnp.bfloat16), jnp.uint16)
    b2 = jax.lax.bitcast_convert_type(b.astype(jnp.bfloat16), jnp.uint16)
    return b1.astype(jnp.int32) | (b2.astype(jnp.int32) << 16)


def _trig_body(rot_ref, hb_ref, out_ref):
    rad = rot_ref[...] * _TWO_PI
    hb = hb_ref[...]
    out_ref[...] = _pack_words(jnp.sin(rad) * hb, jnp.cos(rad) * hb)


def _trig_tables(rotation, has_rotation):
    rot2 = rotation.reshape(_NOBJ // 128, 128)
    hb2 = has_rotation.astype(jnp.float32).reshape(_NOBJ // 128, 128)
    tp = pl.pallas_call(
        _trig_body,
        out_shape=jax.ShapeDtypeStruct((_NOBJ // 128, 128), jnp.int32),
    )(rot2, hb2)
    return tp.reshape(_NOBJ)


def _p_pack_body(p_ref, out_ref):
    for h in range(_H):
        packed = _pack_words(p_ref[0, h, 0], p_ref[0, h, 1])
        a = packed[:, :128]
        b = jnp.concatenate(
            [packed[:, 128:], jnp.zeros((_GY, 96), jnp.int32)], axis=1)
        a3 = a.reshape(_GY // 8, 1, 8, 128)
        b3 = b.reshape(_GY // 8, 1, 8, 128)
        out_ref[0, h] = jnp.concatenate([a3, b3], axis=1).reshape(2 * _GY, 128)


def _pack_predictions(p):
    out = pl.pallas_call(
        _p_pack_body,
        grid=(_B,),
        in_specs=[pl.BlockSpec((1, _H, 2, _GY, _GX),
                               lambda b: (b, 0, 0, 0, 0))],
        out_specs=pl.BlockSpec((1, _H, 2 * _GY, 128),
                               lambda b: (b, 0, 0, 0)),
        out_shape=jax.ShapeDtypeStruct((_B, _H, 2 * _GY, 128), jnp.int32),
    )(p)
    return out.reshape(_NPP)


@functools.partial(
    pl.kernel,
    out_type=jax.ShapeDtypeStruct((_NA // 128, 128), jnp.int32),
    mesh=plsc.VectorSubcoreMesh(core_axis_name="c", subcore_axis_name="s"),
    compiler_params=pltpu.CompilerParams(needs_layout_passes=False),
    scratch_types=[
        pltpu.VMEM((_ROWS_W, 128), jnp.int32),
        pltpu.VMEM((_ROWS_W, 128), jnp.int32),
        pltpu.SemaphoreType.DMA,
    ],
)
def _sc_tgather(tp_hbm, obj_hbm, out_hbm, obj_v, g_v, sem):
    cid = lax.axis_index("c")
    sid = lax.axis_index("s")
    wid = sid * _NC + cid
    row0 = wid * _ROWS_W
    pltpu.async_copy(obj_hbm.at[pl.ds(row0, _ROWS_W)], obj_v, sem).wait()

    def gather_row(r, u):
        pltpu.async_copy(tp_hbm.at[obj_v.at[r]], g_v.at[r], sem)
        return u
    lax.fori_loop(0, _ROWS_W, gather_row, 0)

    def drain_row(r, u):
        pltpu.make_async_copy(tp_hbm.at[pl.ds(0, 128)], g_v.at[r], sem).wait()
        return u
    lax.fori_loop(0, _ROWS_W, drain_row, 0)
    pltpu.sync_copy(g_v, out_hbm.at[pl.ds(row0, _ROWS_W)])


@functools.partial(
    pl.kernel,
    out_type=jax.ShapeDtypeStruct((_NW, 16), jnp.float32),
    mesh=plsc.VectorSubcoreMesh(core_axis_name="c", subcore_axis_name="s"),
    compiler_params=pltpu.CompilerParams(needs_layout_passes=False),
    scratch_types=[
        pltpu.VMEM((2, _SUB, 128), jnp.int32),
        pltpu.VMEM((2, _SUB, 128), jnp.int32),
        pltpu.VMEM((2, _SUB, 128), jnp.int32),
        pltpu.VMEM((2, _SUB, 128), jnp.int32),
        pltpu.VMEM((2, _SUB, 128), jnp.int32),
        pltpu.VMEM((2, _SUB, 128), jnp.int32),
        pltpu.VMEM((2, _SUB, 128), jnp.int32),
        pltpu.VMEM((16,), jnp.float32),
        pltpu.SemaphoreType.DMA,
        pltpu.SemaphoreType.DMA,
    ],
)
def _sc_loss(pp_hbm, tw_hbm, img_hbm, head_hbm, gy_hbm, gx_hbm,
             out_hbm,
             img_v, head_v, gy_v, gx_v, fr_v,
             praw_v, traw_v, res_v, sem_in, sem_g):
    cid = lax.axis_index("c")
    sid = lax.axis_index("s")
    wid = sid * _NC + cid
    row0 = wid * _ROWS_W
    ins = [(img_hbm, img_v), (head_hbm, head_v), (gy_hbm, gy_v),
           (gx_hbm, gx_v), (tw_hbm, traw_v)]
    hi_mask = jnp.full((16,), -65536, jnp.int32)

    def stage_in(t):
        b = t % 2
        r0 = row0 + t * _SUB
        for hbm, v in ins:
            pltpu.async_copy(hbm.at[pl.ds(r0, _SUB)], v.at[b], sem_in)

    def wait_in(t):
        b = t % 2
        for hbm, v in ins:
            pltpu.make_async_copy(hbm.at[pl.ds(0, _SUB)], v.at[b], sem_in).wait()

    def do_idx(t):
        b = t % 2

        def idx_row(r, u):
            for k in range(8):
                sl = pl.ds(k * 16, 16)
                gy = gy_v[b, r, sl]
                gx = gx_v[b, r, sl]
                fr_v[b, r, sl] = (img_v[b, r, sl] * _IMG_STRIDE
                                  + head_v[b, r, sl] * _PLANEP
                                  + ((gy >> 3) << 11) + ((gx >> 7) << 10)
                                  + ((gy & 7) << 7) + (gx & 127))
            return u
        lax.fori_loop(0, _SUB, idx_row, 0)

    def fire(t):
        b = t % 2

        def gather_row(r, u):
            pltpu.async_copy(pp_hbm.at[fr_v.at[b].at[r]],
                             praw_v.at[b].at[r], sem_g)
            return u
        lax.fori_loop(0, _SUB, gather_row, 0)

    def drain(t):
        b = t % 2

        def drain_row(r, u):
            pltpu.make_async_copy(pp_hbm.at[pl.ds(0, 128)],
                                  praw_v.at[b].at[r], sem_g).wait()
            return u
        lax.fori_loop(0, _SUB, drain_row, 0)

    def compute(t, accs):
        b = t % 2

        def comp_row(r, cc):
            a1, a2 = cc
            for k in range(8):
                sl = pl.ds(k * 16, 16)
                pu = praw_v[b, r, sl]
                tu = traw_v[b, r, sl]
                p1 = plsc.bitcast(pu << 16, jnp.float32)
                p2 = plsc.bitcast(pu & hi_mask, jnp.float32)
                sb = plsc.bitcast(tu << 16, jnp.float32)
                cb = plsc.bitcast(tu & hi_mask, jnp.float32)
                bf = sb * sb + cb * cb
                t1 = p1 * sb + p2 * cb - bf
                t2 = p1 * cb - p2 * sb
                a1 = a1 + t1 * t1
                a2 = a2 + t2 * t2
            return (a1, a2)
        return lax.fori_loop(0, _SUB, comp_row, accs)

    stage_in(0)
    wait_in(0)
    do_idx(0)
    fire(0)
    stage_in(1)
    accs = (jnp.zeros((16,), jnp.float32), jnp.zeros((16,), jnp.float32))
    for t in range(_NCHUNK):
        if t + 1 < _NCHUNK:
            wait_in(t + 1)
            do_idx(t + 1)
            fire(t + 1)
        if t + 2 < _NCHUNK:
            stage_in(t + 2)
        drain(t)
        accs = compute(t, accs)
    acc1, acc2 = accs
    res_v[...] = acc1 * _LAM1 + acc2 * _LAM2
    pltpu.sync_copy(res_v, out_hbm.at[wid])


def kernel(post_activation_sincos, rotation, has_rotation, object_idxs,
           img_idxs, head_idxs, grid_y_idxs, grid_x_idxs):
    tpack = _trig_tables(rotation, has_rotation)
    obj2 = object_idxs.reshape(_NA // 128, 128)
    tw = _sc_tgather(tpack, obj2)
    ppack = _pack_predictions(post_activation_sincos)
    img2 = img_idxs.reshape(_NA // 128, 128)
    head2 = head_idxs.reshape(_NA // 128, 128)
    gy2 = grid_y_idxs.reshape(_NA // 128, 128)
    gx2 = grid_x_idxs.reshape(_NA // 128, 128)
    partials = _sc_loss(ppack, tw, img2, head2, gy2, gx2)
    return jnp.sum(partials)


# speedup vs baseline: 2.3924x; 1.2354x over previous
import functools

import jax
import jax.numpy as jnp
from jax import lax
from jax.experimental import pallas as pl
from jax.experimental.pallas import tpu as pltpu
from jax.experimental.pallas import tpu_sc as plsc

_TWO_PI = 2.0 * 3.14159
_ECC = 3.0
_LAM1 = 2.0 / (1.0 + _ECC)
_LAM2 = 2.0 - _LAM1

_B, _H, _GY, _GX = 32, 8, 160, 160
_GXP = 256
_PLANEP = _GY * _GXP
_IMG_STRIDE = _H * _PLANEP
_NOBJ = 262144
_NA = 1048576
_NPP = _B * _H * _PLANEP

_NW = 32
_NC = 2
_PER_W = _NA // _NW
_CHUNK = 8192
_SUB = _CHUNK // 128
_NCHUNK = _PER_W // _CHUNK
_ROWS_W = _PER_W // 128


def _pack_words(a, b):
    b1 = jax.lax.bitcast_convert_type(a.astype(jnp.bfloat16), jnp.uint16)
    b2 = jax.lax.bitcast_convert_type(b.astype(jnp.bfloat16), jnp.uint16)
    return b1.astype(jnp.int32) | (b2.astype(jnp.int32) << 16)


def _trig_body(rot_ref, hb_ref, out_ref):
    rad = rot_ref[...] * _TWO_PI
    hb = hb_ref[...]
    out_ref[...] = _pack_words(jnp.sin(rad) * hb, jnp.cos(rad) * hb)


def _trig_tables(rotation, has_rotation):
    rot2 = rotation.reshape(_NOBJ // 128, 128)
    hb2 = has_rotation.astype(jnp.float32).reshape(_NOBJ // 128, 128)
    tp = pl.pallas_call(
        _trig_body,
        out_shape=jax.ShapeDtypeStruct((_NOBJ // 128, 128), jnp.int32),
    )(rot2, hb2)
    return tp.reshape(_NOBJ)


def _p_pack_body(p_ref, out_ref):
    for h in range(_H):
        packed = _pack_words(p_ref[0, h, 0], p_ref[0, h, 1])
        a = packed[:, :128]
        b = jnp.concatenate(
            [packed[:, 128:], jnp.zeros((_GY, 96), jnp.int32)], axis=1)
        a3 = a.reshape(_GY // 8, 1, 8, 128)
        b3 = b.reshape(_GY // 8, 1, 8, 128)
        out_ref[0, h] = jnp.concatenate([a3, b3], axis=1).reshape(2 * _GY, 128)


def _pack_predictions(p):
    out = pl.pallas_call(
        _p_pack_body,
        grid=(_B,),
        in_specs=[pl.BlockSpec((1, _H, 2, _GY, _GX),
                               lambda b: (b, 0, 0, 0, 0))],
        out_specs=pl.BlockSpec((1, _H, 2 * _GY, 128),
                               lambda b: (b, 0, 0, 0)),
        out_shape=jax.ShapeDtypeStruct((_B, _H, 2 * _GY, 128), jnp.int32),
    )(p)
    return out.reshape(_NPP)


@functools.partial(
    pl.kernel,
    out_type=jax.ShapeDtypeStruct((_NA // 128, 128), jnp.int32),
    mesh=plsc.VectorSubcoreMesh(core_axis_name="c", subcore_axis_name="s"),
    compiler_params=pltpu.CompilerParams(needs_layout_passes=False),
    scratch_types=[
        pltpu.VMEM((_ROWS_W, 128), jnp.int32),
        pltpu.VMEM((_ROWS_W, 128), jnp.int32),
        pltpu.VMEM_SHARED((_NOBJ,), jnp.int32),
        pltpu.SemaphoreType.DMA,
    ],
)
def _sc_tgather(tp_hbm, obj_hbm, out_hbm, obj_v, g_v, tp_sh, sem):
    cid = lax.axis_index("c")
    sid = lax.axis_index("s")
    wid = sid * _NC + cid
    row0 = wid * _ROWS_W
    seg = _NOBJ // 16
    c0 = pltpu.async_copy(obj_hbm.at[pl.ds(row0, _ROWS_W)], obj_v, sem)
    pltpu.sync_copy(tp_hbm.at[pl.ds(sid * seg, seg)],
                    tp_sh.at[pl.ds(sid * seg, seg)])
    plsc.subcore_barrier()
    c0.wait()

    def gather_row(r, u):
        pltpu.async_copy(tp_sh.at[obj_v.at[r]], g_v.at[r], sem)
        return u
    lax.fori_loop(0, _ROWS_W, gather_row, 0)

    def drain_row(r, u):
        pltpu.make_async_copy(tp_hbm.at[pl.ds(0, 128)], g_v.at[r], sem).wait()
        return u
    lax.fori_loop(0, _ROWS_W, drain_row, 0)
    pltpu.sync_copy(g_v, out_hbm.at[pl.ds(row0, _ROWS_W)])


@functools.partial(
    pl.kernel,
    out_type=jax.ShapeDtypeStruct((_NW, 16), jnp.float32),
    mesh=plsc.VectorSubcoreMesh(core_axis_name="c", subcore_axis_name="s"),
    compiler_params=pltpu.CompilerParams(needs_layout_passes=False),
    scratch_types=[
        pltpu.VMEM((2, _SUB, 128), jnp.int32),
        pltpu.VMEM((2, _SUB, 128), jnp.int32),
        pltpu.VMEM((2, _SUB, 128), jnp.int32),
        pltpu.VMEM((2, _SUB, 128), jnp.int32),
        pltpu.VMEM((2, _SUB, 128), jnp.int32),
        pltpu.VMEM((2, _SUB, 128), jnp.int32),
        pltpu.VMEM((2, _SUB, 128), jnp.int32),
        pltpu.VMEM((16,), jnp.float32),
        pltpu.SemaphoreType.DMA,
        pltpu.SemaphoreType.DMA,
    ],
)
def _sc_loss(pp_hbm, tw_hbm, img_hbm, head_hbm, gy_hbm, gx_hbm,
             out_hbm,
             img_v, head_v, gy_v, gx_v, fr_v,
             praw_v, traw_v, res_v, sem_in, sem_g):
    cid = lax.axis_index("c")
    sid = lax.axis_index("s")
    wid = sid * _NC + cid
    row0 = wid * _ROWS_W
    ins = [(img_hbm, img_v), (head_hbm, head_v), (gy_hbm, gy_v),
           (gx_hbm, gx_v), (tw_hbm, traw_v)]
    hi_mask = jnp.full((16,), -65536, jnp.int32)

    def stage_in(t):
        b = t % 2
        r0 = row0 + t * _SUB
        for hbm, v in ins:
            pltpu.async_copy(hbm.at[pl.ds(r0, _SUB)], v.at[b], sem_in)

    def wait_in(t):
        b = t % 2
        for hbm, v in ins:
            pltpu.make_async_copy(hbm.at[pl.ds(0, _SUB)], v.at[b], sem_in).wait()

    def do_idx(t):
        b = t % 2

        def idx_row(r, u):
            for k in range(8):
                sl = pl.ds(k * 16, 16)
                gy = gy_v[b, r, sl]
                gx = gx_v[b, r, sl]
                fr_v[b, r, sl] = (img_v[b, r, sl] * _IMG_STRIDE
                                  + head_v[b, r, sl] * _PLANEP
                                  + ((gy >> 3) << 11) + ((gx >> 7) << 10)
                                  + ((gy & 7) << 7) + (gx & 127))
            return u
        lax.fori_loop(0, _SUB, idx_row, 0)

    def fire(t):
        b = t % 2

        def gather_row(r, u):
            pltpu.async_copy(pp_hbm.at[fr_v.at[b].at[r]],
                             praw_v.at[b].at[r], sem_g)
            return u
        lax.fori_loop(0, _SUB, gather_row, 0)

    def drain(t):
        b = t % 2

        def drain_row(r, u):
            pltpu.make_async_copy(pp_hbm.at[pl.ds(0, 128)],
                                  praw_v.at[b].at[r], sem_g).wait()
            return u
        lax.fori_loop(0, _SUB, drain_row, 0)

    def compute(t, accs):
        b = t % 2

        def comp_row(r, cc):
            a1, a2 = cc
            for k in range(8):
                sl = pl.ds(k * 16, 16)
                pu = praw_v[b, r, sl]
                tu = traw_v[b, r, sl]
                p1 = plsc.bitcast(pu << 16, jnp.float32)
                p2 = plsc.bitcast(pu & hi_mask, jnp.float32)
                sb = plsc.bitcast(tu << 16, jnp.float32)
                cb = plsc.bitcast(tu & hi_mask, jnp.float32)
                bf = sb * sb + cb * cb
                t1 = p1 * sb + p2 * cb - bf
                t2 = p1 * cb - p2 * sb
                a1 = a1 + t1 * t1
                a2 = a2 + t2 * t2
            return (a1, a2)
        return lax.fori_loop(0, _SUB, comp_row, accs)

    stage_in(0)
    wait_in(0)
    do_idx(0)
    fire(0)
    stage_in(1)
    accs = (jnp.zeros((16,), jnp.float32), jnp.zeros((16,), jnp.float32))
    for t in range(_NCHUNK):
        if t + 1 < _NCHUNK:
            wait_in(t + 1)
            do_idx(t + 1)
            fire(t + 1)
        if t + 2 < _NCHUNK:
            stage_in(t + 2)
        drain(t)
        accs = compute(t, accs)
    acc1, acc2 = accs
    res_v[...] = acc1 * _LAM1 + acc2 * _LAM2
    pltpu.sync_copy(res_v, out_hbm.at[wid])


def kernel(post_activation_sincos, rotation, has_rotation, object_idxs,
           img_idxs, head_idxs, grid_y_idxs, grid_x_idxs):
    tpack = _trig_tables(rotation, has_rotation)
    obj2 = object_idxs.reshape(_NA // 128, 128)
    tw = _sc_tgather(tpack, obj2)
    ppack = _pack_predictions(post_activation_sincos)
    img2 = img_idxs.reshape(_NA // 128, 128)
    head2 = head_idxs.reshape(_NA // 128, 128)
    gy2 = grid_y_idxs.reshape(_NA // 128, 128)
    gx2 = grid_x_idxs.reshape(_NA // 128, 128)
    partials = _sc_loss(ppack, tw, img2, head2, gy2, gx2)
    return jnp.sum(partials)


# speedup vs baseline: 2.4281x; 1.0149x over previous
import functools

import jax
import jax.numpy as jnp
from jax import lax
from jax.experimental import pallas as pl
from jax.experimental.pallas import tpu as pltpu
from jax.experimental.pallas import tpu_sc as plsc

_TWO_PI = 2.0 * 3.14159
_ECC = 3.0
_LAM1 = 2.0 / (1.0 + _ECC)
_LAM2 = 2.0 - _LAM1

_B, _H, _GY, _GX = 32, 8, 160, 160
_PLANEW = 200 * 128
_TAIL0 = _GY * 128
_IMG_STRIDE = _H * _PLANEW
_NOBJ = 262144
_NA = 1048576
_NPP = _B * _H * _PLANEW

_NW = 32
_NC = 2
_PER_W = _NA // _NW
_CHUNK = 8192
_SUB = _CHUNK // 128
_NCHUNK = _PER_W // _CHUNK
_ROWS_W = _PER_W // 128


def _pack_words(a, b):
    b1 = jax.lax.bitcast_convert_type(a.astype(jnp.bfloat16), jnp.uint16)
    b2 = jax.lax.bitcast_convert_type(b.astype(jnp.bfloat16), jnp.uint16)
    return b1.astype(jnp.int32) | (b2.astype(jnp.int32) << 16)


def _trig_body(rot_ref, hb_ref, out_ref):
    rad = rot_ref[...] * _TWO_PI
    hb = hb_ref[...]
    out_ref[...] = _pack_words(jnp.sin(rad) * hb, jnp.cos(rad) * hb)


def _trig_tables(rotation, has_rotation):
    rot2 = rotation.reshape(_NOBJ // 128, 128)
    hb2 = has_rotation.astype(jnp.float32).reshape(_NOBJ // 128, 128)
    tp = pl.pallas_call(
        _trig_body,
        out_shape=jax.ShapeDtypeStruct((_NOBJ // 128, 128), jnp.int32),
    )(rot2, hb2)
    return tp.reshape(_NOBJ)


def _p_pack_body(p_ref, out_ref):
    for h in range(_H):
        packed = _pack_words(p_ref[0, h, 0], p_ref[0, h, 1])
        a = packed[:, :128]
        t = packed[:, 128:]
        tiles = []
        for k in range(5):
            pieces = [t[(4 * k + j) * 8:(4 * k + j + 1) * 8, :]
                      for j in range(4)]
            tiles.append(jnp.concatenate(pieces, axis=1))
        tail = jnp.concatenate(tiles, axis=0)
        out_ref[0, h] = jnp.concatenate([a, tail], axis=0)


def _pack_predictions(p):
    out = pl.pallas_call(
        _p_pack_body,
        grid=(_B,),
        in_specs=[pl.BlockSpec((1, _H, 2, _GY, _GX),
                               lambda b: (b, 0, 0, 0, 0))],
        out_specs=pl.BlockSpec((1, _H, 200, 128),
                               lambda b: (b, 0, 0, 0)),
        out_shape=jax.ShapeDtypeStruct((_B, _H, 200, 128), jnp.int32),
    )(p)
    return out.reshape(_NPP)


@functools.partial(
    pl.kernel,
    out_type=jax.ShapeDtypeStruct((_NA // 128, 128), jnp.int32),
    mesh=plsc.VectorSubcoreMesh(core_axis_name="c", subcore_axis_name="s"),
    compiler_params=pltpu.CompilerParams(needs_layout_passes=False),
    scratch_types=[
        pltpu.VMEM((_ROWS_W, 128), jnp.int32),
        pltpu.VMEM((_ROWS_W, 128), jnp.int32),
        pltpu.VMEM_SHARED((_NOBJ,), jnp.int32),
        pltpu.SemaphoreType.DMA,
    ],
)
def _sc_tgather(tp_hbm, obj_hbm, out_hbm, obj_v, g_v, tp_sh, sem):
    cid = lax.axis_index("c")
    sid = lax.axis_index("s")
    wid = sid * _NC + cid
    row0 = wid * _ROWS_W
    seg = _NOBJ // 16
    c0 = pltpu.async_copy(obj_hbm.at[pl.ds(row0, _ROWS_W)], obj_v, sem)
    pltpu.sync_copy(tp_hbm.at[pl.ds(sid * seg, seg)],
                    tp_sh.at[pl.ds(sid * seg, seg)])
    plsc.subcore_barrier()
    c0.wait()

    def gather_row(r, u):
        pltpu.async_copy(tp_sh.at[obj_v.at[r]], g_v.at[r], sem)
        return u
    lax.fori_loop(0, _ROWS_W, gather_row, 0)

    def drain_row(r, u):
        pltpu.make_async_copy(tp_hbm.at[pl.ds(0, 128)], g_v.at[r], sem).wait()
        return u
    lax.fori_loop(0, _ROWS_W, drain_row, 0)
    pltpu.sync_copy(g_v, out_hbm.at[pl.ds(row0, _ROWS_W)])


@functools.partial(
    pl.kernel,
    out_type=jax.ShapeDtypeStruct((_NW, 16), jnp.float32),
    mesh=plsc.VectorSubcoreMesh(core_axis_name="c", subcore_axis_name="s"),
    compiler_params=pltpu.CompilerParams(needs_layout_passes=False),
    scratch_types=[
        pltpu.VMEM((2, _SUB, 128), jnp.int32),
        pltpu.VMEM((2, _SUB, 128), jnp.int32),
        pltpu.VMEM((2, _SUB, 128), jnp.int32),
        pltpu.VMEM((2, _SUB, 128), jnp.int32),
        pltpu.VMEM((2, _SUB, 128), jnp.int32),
        pltpu.VMEM((2, _SUB, 128), jnp.int32),
        pltpu.VMEM((2, _SUB, 128), jnp.int32),
        pltpu.VMEM((16,), jnp.float32),
        pltpu.SemaphoreType.DMA,
        pltpu.SemaphoreType.DMA,
    ],
)
def _sc_loss(pp_hbm, tw_hbm, img_hbm, head_hbm, gy_hbm, gx_hbm,
             out_hbm,
             img_v, head_v, gy_v, gx_v, fr_v,
             praw_v, traw_v, res_v, sem_in, sem_g):
    cid = lax.axis_index("c")
    sid = lax.axis_index("s")
    wid = sid * _NC + cid
    row0 = wid * _ROWS_W
    ins = [(img_hbm, img_v), (head_hbm, head_v), (gy_hbm, gy_v),
           (gx_hbm, gx_v), (tw_hbm, traw_v)]
    hi_mask = jnp.full((16,), -65536, jnp.int32)

    def stage_in(t):
        b = t % 2
        r0 = row0 + t * _SUB
        for hbm, v in ins:
            pltpu.async_copy(hbm.at[pl.ds(r0, _SUB)], v.at[b], sem_in)

    def wait_in(t):
        b = t % 2
        for hbm, v in ins:
            pltpu.make_async_copy(hbm.at[pl.ds(0, _SUB)], v.at[b], sem_in).wait()

    def do_idx(t):
        b = t % 2

        def idx_row(r, u):
            for k in range(8):
                sl = pl.ds(k * 16, 16)
                gy = gy_v[b, r, sl]
                gx = gx_v[b, r, sl]
                plane = (img_v[b, r, sl] * _IMG_STRIDE
                         + head_v[b, r, sl] * _PLANEW)
                addr_a = plane + (gy << 7) + gx
                band = gy >> 3
                addr_t = (plane + _TAIL0 + ((band >> 2) << 10)
                          + ((gy & 7) << 7) + ((band & 3) << 5) + (gx & 127))
                fr_v[b, r, sl] = jnp.where(gx < 128, addr_a, addr_t)
            return u
        lax.fori_loop(0, _SUB, idx_row, 0)

    def fire(t):
        b = t % 2

        def gather_row(r, u):
            pltpu.async_copy(pp_hbm.at[fr_v.at[b].at[r]],
                             praw_v.at[b].at[r], sem_g)
            return u
        lax.fori_loop(0, _SUB, gather_row, 0)

    def drain(t):
        b = t % 2

        def drain_row(r, u):
            pltpu.make_async_copy(pp_hbm.at[pl.ds(0, 128)],
                                  praw_v.at[b].at[r], sem_g).wait()
            return u
        lax.fori_loop(0, _SUB, drain_row, 0)

    def compute(t, accs):
        b = t % 2

        def comp_row(r, cc):
            a1, a2 = cc
            for k in range(8):
                sl = pl.ds(k * 16, 16)
                pu = praw_v[b, r, sl]
                tu = traw_v[b, r, sl]
                p1 = plsc.bitcast(pu << 16, jnp.float32)
                p2 = plsc.bitcast(pu & hi_mask, jnp.float32)
                sb = plsc.bitcast(tu << 16, jnp.float32)
                cb = plsc.bitcast(tu & hi_mask, jnp.float32)
                bf = sb * sb + cb * cb
                t1 = p1 * sb + p2 * cb - bf
                t2 = p1 * cb - p2 * sb
                a1 = a1 + t1 * t1
                a2 = a2 + t2 * t2
            return (a1, a2)
        return lax.fori_loop(0, _SUB, comp_row, accs)

    stage_in(0)
    wait_in(0)
    do_idx(0)
    fire(0)
    stage_in(1)
    accs = (jnp.zeros((16,), jnp.float32), jnp.zeros((16,), jnp.float32))
    for t in range(_NCHUNK):
        if t + 1 < _NCHUNK:
            wait_in(t + 1)
            do_idx(t + 1)
            fire(t + 1)
        if t + 2 < _NCHUNK:
            stage_in(t + 2)
        drain(t)
        accs = compute(t, accs)
    acc1, acc2 = accs
    res_v[...] = acc1 * _LAM1 + acc2 * _LAM2
    pltpu.sync_copy(res_v, out_hbm.at[wid])


def kernel(post_activation_sincos, rotation, has_rotation, object_idxs,
           img_idxs, head_idxs, grid_y_idxs, grid_x_idxs):
    tpack = _trig_tables(rotation, has_rotation)
    obj2 = object_idxs.reshape(_NA // 128, 128)
    tw = _sc_tgather(tpack, obj2)
    ppack = _pack_predictions(post_activation_sincos)
    img2 = img_idxs.reshape(_NA // 128, 128)
    head2 = head_idxs.reshape(_NA // 128, 128)
    gy2 = grid_y_idxs.reshape(_NA // 128, 128)
    gx2 = grid_x_idxs.reshape(_NA // 128, 128)
    partials = _sc_loss(ppack, tw, img2, head2, gy2, gx2)
    return jnp.sum(partials)
